# baseline (speedup 1.0000x reference)
import jax
import jax.numpy as jnp
from jax import lax
from jax.experimental import pallas as pl
from jax.experimental.pallas import tpu as pltpu
from jax.experimental.pallas import tpu_sc as plsc

L = 16
N = 4
C = 384
H = 224
W = 224
PIX = N * H * W
NC = 2
NS = 16
NW = NC * NS
PPW = PIX // NW
GCHUNK = 32
SCHUNK = 112
SBUF = 2
SROWS = 4480


def _sc_body(inp_hbm, mask_hbm, out_hbm, mask_v, sel_v, shared_v,
             buf0, buf1, isem0, isem1, osem0, osem1,
             isem2, isem3, osem2, osem3):
    cid = lax.axis_index("c")
    sid = lax.axis_index("s")
    wid = sid * NC + cid

    pltpu.sync_copy(mask_hbm, mask_v)
    zeros = jnp.zeros((L,), jnp.int32)
    for k in range(C // L):
        sel_v[pl.ds(k * L, L)] = zeros
    count = jnp.int32(0)
    mism = jnp.int32(0)
    for k in range(C // L):
        v = mask_v[pl.ds(k * L, L)]
        nz = v != 0.0
        nzi = nz.astype(jnp.int32)
        cs = plsc.cumsum(nzi)
        pos = count + cs - nzi
        vals = lax.iota(jnp.int32, L) + (k * L)
        plsc.store_scatter(sel_v, [pos], vals, mask=nz)
        count = count + jnp.sum(nzi)
        mism = mism + jnp.sum((~nz).astype(jnp.int32))

    base = wid * PPW
    in_sems = (isem0, isem1)
    out_sems = (osem0, osem1)
    nq = PPW // GCHUNK

    @pl.when(mism == 0)
    def _fast():
        snq = SROWS // SCHUNK
        tnq = (PPW - SROWS) // GCHUNK
        tbase = base + SROWS
        tbufs = (buf0, buf1)
        tin = (isem2, isem3)
        tout = (osem2, osem3)

        def s_read(q):
            return pltpu.async_copy(
                inp_hbm.at[pl.ds(base + q * SCHUNK, SCHUNK)],
                shared_v.at[sid, q % SBUF], in_sems[q % SBUF])

        def s_write(q):
            return pltpu.async_copy(
                shared_v.at[sid, q % SBUF],
                out_hbm.at[pl.ds(base + q * SCHUNK, SCHUNK)],
                out_sems[q % SBUF])

        def t_read(q):
            return pltpu.async_copy(
                inp_hbm.at[pl.ds(tbase + q * GCHUNK, GCHUNK)],
                tbufs[q % 2], tin[q % 2])

        def t_write(q):
            return pltpu.async_copy(
                tbufs[q % 2],
                out_hbm.at[pl.ds(tbase + q * GCHUNK, GCHUNK)],
                tout[q % 2])

        def run_ring(n, start_r, start_w, depth):
            reads = [None] * n
            writes = [None] * n
            for q in range(min(depth - 1, n)):
                reads[q] = start_r(q)
            for q in range(n):
                if q + depth - 1 < n:
                    if q - 1 >= 0:
                        writes[q - 1].wait()
                    reads[q + depth - 1] = start_r(q + depth - 1)
                reads[q].wait()
                writes[q] = start_w(q)
                yield
            for r in range(max(0, n - depth), n):
                writes[r].wait()

        ra = run_ring(snq, s_read, s_write, SBUF)
        rb = run_ring(tnq, t_read, t_write, 2)
        alive = [ra, rb]
        while alive:
            for g in list(alive):
                try:
                    next(g)
                except StopIteration:
                    alive.remove(g)

    @pl.when(mism != 0)
    def _general():
        def chunk_body(q, carry):
            lo = base + q * GCHUNK
            pltpu.sync_copy(inp_hbm.at[pl.ds(lo, GCHUNK)], buf0)

            def pixel_body(p, c2):
                for g in range(C // L):
                    off = pl.multiple_of(g * L, 8)
                    src_c = sel_v[pl.ds(off, L)]
                    rows = jnp.zeros((L,), jnp.int32) + p
                    vals = plsc.load_gather(buf0, [rows, src_c])
                    buf1[p, pl.ds(g * L, L)] = vals
                return c2
            lax.fori_loop(0, GCHUNK, pixel_body, jnp.int32(0))

            pltpu.sync_copy(buf1, out_hbm.at[pl.ds(lo, GCHUNK)])
            return carry
        lax.fori_loop(0, nq, chunk_body, jnp.int32(0))


@jax.jit
def _sc_gather(inp2, mask):
    mesh = plsc.VectorSubcoreMesh(core_axis_name="c", subcore_axis_name="s",
                                  num_cores=NC, num_subcores=NS)
    return pl.kernel(
        _sc_body,
        out_type=jax.ShapeDtypeStruct((PIX, C), jnp.float32),
        mesh=mesh,
        compiler_params=pltpu.CompilerParams(needs_layout_passes=False),
        scratch_types=[
            pltpu.VMEM((C,), jnp.float32),
            pltpu.VMEM((C,), jnp.int32),
            pltpu.VMEM_SHARED((NS, SBUF, SCHUNK, C), jnp.float32),
            pltpu.VMEM((GCHUNK, C), jnp.float32),
            pltpu.VMEM((GCHUNK, C), jnp.float32),
            pltpu.SemaphoreType.DMA,
            pltpu.SemaphoreType.DMA,
            pltpu.SemaphoreType.DMA,
            pltpu.SemaphoreType.DMA,
            pltpu.SemaphoreType.DMA,
            pltpu.SemaphoreType.DMA,
            pltpu.SemaphoreType.DMA,
            pltpu.SemaphoreType.DMA,
        ],
    )(inp2, mask)


def kernel(input_tensor, indexes):
    x2 = input_tensor.transpose(0, 2, 3, 1).reshape(PIX, C)
    out2 = _sc_gather(x2, indexes)
    return out2.reshape(N, H, W, C).transpose(0, 3, 1, 2)


# speedup vs baseline: 1.0489x; 1.0489x over previous
import jax
import jax.numpy as jnp
from jax import lax
from jax.experimental import pallas as pl
from jax.experimental.pallas import tpu as pltpu
from jax.experimental.pallas import tpu_sc as plsc

L = 16
N = 4
C = 384
H = 224
W = 224
PIX = N * H * W
NC = 2
NS = 16
NW = NC * NS
PPW = PIX // NW
GCHUNK = 32
SCHUNK = 112
SBUF = 2


def _sc_body(inp_hbm, mask_hbm, out_hbm, mask_v, sel_v, shared_v,
             buf0, buf1, isem0, isem1, osem0, osem1):
    cid = lax.axis_index("c")
    sid = lax.axis_index("s")
    wid = sid * NC + cid

    pltpu.sync_copy(mask_hbm, mask_v)
    zeros = jnp.zeros((L,), jnp.int32)
    for k in range(C // L):
        sel_v[pl.ds(k * L, L)] = zeros
    count = jnp.int32(0)
    mism = jnp.int32(0)
    for k in range(C // L):
        v = mask_v[pl.ds(k * L, L)]
        nz = v != 0.0
        nzi = nz.astype(jnp.int32)
        cs = plsc.cumsum(nzi)
        pos = count + cs - nzi
        vals = lax.iota(jnp.int32, L) + (k * L)
        plsc.store_scatter(sel_v, [pos], vals, mask=nz)
        count = count + jnp.sum(nzi)
        mism = mism + jnp.sum((~nz).astype(jnp.int32))

    base = wid * PPW
    in_sems = (isem0, isem1)
    out_sems = (osem0, osem1)
    nq = PPW // GCHUNK

    @pl.when(mism == 0)
    def _fast():
        snq = PPW // SCHUNK

        def s_read(q):
            return pltpu.async_copy(
                inp_hbm.at[pl.ds(base + q * SCHUNK, SCHUNK)],
                shared_v.at[sid, q % SBUF], in_sems[q % SBUF])

        def s_write(q):
            return pltpu.async_copy(
                shared_v.at[sid, q % SBUF],
                out_hbm.at[pl.ds(base + q * SCHUNK, SCHUNK)],
                out_sems[q % SBUF])

        reads = [None] * snq
        writes = [None] * snq
        for q in range(min(SBUF - 1, snq)):
            reads[q] = s_read(q)
        for q in range(snq):
            if q + SBUF - 1 < snq:
                if q - 1 >= 0:
                    writes[q - 1].wait()
                reads[q + SBUF - 1] = s_read(q + SBUF - 1)
            reads[q].wait()
            writes[q] = s_write(q)
        for r in range(max(0, snq - SBUF), snq):
            writes[r].wait()

    @pl.when(mism != 0)
    def _general():
        def chunk_body(q, carry):
            lo = base + q * GCHUNK
            pltpu.sync_copy(inp_hbm.at[pl.ds(lo, GCHUNK)], buf0)

            def pixel_body(p, c2):
                for g in range(C // L):
                    off = pl.multiple_of(g * L, 8)
                    src_c = sel_v[pl.ds(off, L)]
                    rows = jnp.zeros((L,), jnp.int32) + p
                    vals = plsc.load_gather(buf0, [rows, src_c])
                    buf1[p, pl.ds(g * L, L)] = vals
                return c2
            lax.fori_loop(0, GCHUNK, pixel_body, jnp.int32(0))

            pltpu.sync_copy(buf1, out_hbm.at[pl.ds(lo, GCHUNK)])
            return carry
        lax.fori_loop(0, nq, chunk_body, jnp.int32(0))


@jax.jit
def _sc_gather(inp2, mask):
    mesh = plsc.VectorSubcoreMesh(core_axis_name="c", subcore_axis_name="s",
                                  num_cores=NC, num_subcores=NS)
    return pl.kernel(
        _sc_body,
        out_type=jax.ShapeDtypeStruct((PIX, C), jnp.float32),
        mesh=mesh,
        compiler_params=pltpu.CompilerParams(needs_layout_passes=False),
        scratch_types=[
            pltpu.VMEM((C,), jnp.float32),
            pltpu.VMEM((C,), jnp.int32),
            pltpu.VMEM_SHARED((NS, SBUF, SCHUNK, C), jnp.float32),
            pltpu.VMEM((GCHUNK, C), jnp.float32),
            pltpu.VMEM((GCHUNK, C), jnp.float32),
            pltpu.SemaphoreType.DMA,
            pltpu.SemaphoreType.DMA,
            pltpu.SemaphoreType.DMA,
            pltpu.SemaphoreType.DMA,
        ],
    )(inp2, mask)


def kernel(input_tensor, indexes):
    x2 = input_tensor.transpose(0, 2, 3, 1).reshape(PIX, C)
    out2 = _sc_gather(x2, indexes)
    return out2.reshape(N, H, W, C).transpose(0, 3, 1, 2)
